# baseline (speedup 1.0000x reference)
import functools

import jax
import jax.numpy as jnp
from jax import lax
from jax.experimental import pallas as pl
from jax.experimental.pallas import tpu as pltpu
from jax.experimental.pallas import tpu_sc as plsc

NUM_CORES = 2
NUM_SUBCORES = 16
LANES = 16
NUM_WORKERS = NUM_CORES * NUM_SUBCORES

BATCH = 16384
D = 64
BPW = BATCH // NUM_WORKERS
NVEC = BATCH // LANES
NBKT = 256
HITCAP = 1024
BLK = 128

NUSERS = 1000000
NMOVIES = 100000

_mesh = plsc.VectorSubcoreMesh(core_axis_name="c", subcore_axis_name="s")


def _extract_one(table_hbm, idx_hbm, out_hbm, n_rows,
                 idxbuf_v, cnt_v, cur_v, hits_v, sorted_v, sorted_s, starts_s,
                 blockbuf_v, stage_v, sem_blk, sem_out, wid):
    max_j = (n_rows - 1) // BLK
    nloc = max_j // NUM_WORKERS + 2

    lane = lax.iota(jnp.int32, LANES)
    zeros16 = jnp.zeros((LANES,), jnp.int32)
    ones16 = jnp.ones((LANES,), jnp.int32)

    pltpu.sync_copy(idx_hbm, idxbuf_v)

    for k in range(NBKT // LANES):
        cnt_v[pl.ds(k * LANES, LANES)] = zeros16

    def compact_body(v4, cursor):
        for i in range(8):
            v = v4 * 8 + i
            u = idxbuf_v[pl.ds(v * LANES, LANES)]
            j = lax.shift_right_logical(u, 7)
            own = lax.bitwise_and(j, NUM_WORKERS - 1) == wid
            l = lax.shift_right_logical(j, 5)
            rl = lax.bitwise_and(u, BLK - 1)
            b_vec = lane + v * LANES
            packed = lax.bitwise_or(
                lax.bitwise_or(lax.shift_left(b_vec, 15),
                               lax.shift_left(l, 7)), rl)
            plsc.store_compressed(hits_v.at[pl.ds(cursor, LANES)], packed, mask=own)
            cnt = plsc.all_reduce_population_count(own)
            cursor = cursor + cnt[0]
        return cursor

    total_hits = lax.fori_loop(0, NVEC // 8, compact_body, jnp.int32(0))
    nhv = lax.div(total_hits + (LANES - 1), jnp.int32(LANES))

    def count_body(v, carry):
        p = hits_v[pl.ds(v * LANES, LANES)]
        valid = (lane + v * LANES) < total_hits
        l = lax.bitwise_and(lax.shift_right_logical(p, 7), NBKT - 1)
        rank, last = plsc.scan_count(l, mask=valid)
        plsc.addupdate_scatter(cnt_v, [l], rank,
                               mask=lax.bitwise_and(last, valid))
        return carry

    lax.fori_loop(0, nhv, count_body, 0)

    carry = jnp.int32(0)
    for k in range(NBKT // LANES):
        sl = pl.ds(k * LANES, LANES)
        c = cnt_v[sl]
        cum = plsc.cumsum(c)
        excl = cum - c + carry
        cur_v[sl] = excl
        for j in range(LANES):
            starts_s[k * LANES + j] = excl[j]
        carry = carry + cum[LANES - 1]
    starts_s[NBKT] = total_hits

    def scat_body(v, carry):
        p = hits_v[pl.ds(v * LANES, LANES)]
        valid = (lane + v * LANES) < total_hits
        l = lax.bitwise_and(lax.shift_right_logical(p, 7), NBKT - 1)
        rank, last = plsc.scan_count(l, mask=valid)
        base = plsc.load_gather(cur_v, [l])
        pos = base + rank - 1
        b_vec = lax.shift_right_logical(p, 15)
        p2 = lax.bitwise_or(lax.shift_left(b_vec, 7),
                            lax.bitwise_and(p, BLK - 1))
        plsc.store_scatter(sorted_v, [pos], p2, mask=valid)
        plsc.store_scatter(cur_v, [l], pos + ones16,
                           mask=lax.bitwise_and(last, valid))
        return carry

    lax.fori_loop(0, nhv, scat_body, 0)

    def smem_body(v, carry):
        s = sorted_v[pl.ds(v * LANES, LANES)]
        for j in range(LANES):
            sorted_s[v * LANES + j] = s[j]
        return carry

    lax.fori_loop(0, nhv, smem_body, 0)

    def nonempty(l):
        lo = starts_s[l]
        hi = jnp.where(l + 1 < NBKT, starts_s[l + 1], total_hits)
        return hi > lo

    def fetch(l):
        j = l * NUM_WORKERS + wid
        par = lax.rem(l, 12)

        @pl.when(jnp.logical_and(
            jnp.logical_and(j <= max_j, l < nloc), nonempty(l)))
        def _():
            off = pl.multiple_of(j * BLK, BLK)
            row = pl.multiple_of(par * D, 8)
            pltpu.async_copy(
                table_hbm.at[pl.ds(0, D), pl.ds(off, BLK)],
                blockbuf_v.at[pl.ds(row, D), pl.ds(0, BLK)], sem_blk)

    for p in range(11):
        fetch(jnp.int32(p))

    def sweep_body(l, carry):
        j = l * NUM_WORKERS + wid
        par = lax.rem(l, 12)
        inflight = carry

        @pl.when(jnp.logical_and(j <= max_j, nonempty(l)))
        def _():
            row = pl.multiple_of(par * D, 8)
            pltpu.make_async_copy(
                table_hbm.at[pl.ds(0, D), pl.ds(0, BLK)],
                blockbuf_v.at[pl.ds(row, D), pl.ds(0, BLK)], sem_blk).wait()

        fetch(l + 11)

        def hit_body(h, infl, par=par):
            s = sorted_s[h]
            b = lax.shift_right_logical(s, 7)
            rl = lax.bitwise_and(s, BLK - 1)
            slot = lax.rem(h, 16)

            @pl.when(infl >= 16)
            def _():
                pltpu.make_async_copy(
                    stage_v.at[pl.ds(0, D)], out_hbm.at[pl.ds(0, D)],
                    sem_out).wait()

            rowbase = lane + par * D
            for k in range(D // LANES):
                col = plsc.load_gather(
                    blockbuf_v,
                    [rowbase + k * LANES,
                     jnp.full((LANES,), 0, jnp.int32) + rl])
                stage_v[pl.ds(slot * D + k * LANES, LANES)] = col
            pltpu.async_copy(stage_v.at[pl.ds(slot * D, D)],
                             out_hbm.at[pl.ds(b * D, D)], sem_out)
            return infl + 1

        lo = starts_s[l]
        hi = jnp.where(l + 1 < NBKT, starts_s[l + 1], total_hits)
        hi = jnp.where(j <= max_j, hi, lo)
        inflight = lax.fori_loop(lo, hi, hit_body, inflight)
        return inflight

    inflight = lax.fori_loop(0, nloc, sweep_body, jnp.int32(0))

    def drain_body(i, carry):
        pltpu.make_async_copy(
            stage_v.at[pl.ds(0, D)], out_hbm.at[pl.ds(0, D)], sem_out).wait()
        return carry

    lax.fori_loop(0, lax.min(inflight, jnp.int32(16)), drain_body, 0)


@functools.partial(
    pl.kernel,
    out_type=(jax.ShapeDtypeStruct((BATCH * D,), jnp.float32),
              jax.ShapeDtypeStruct((BATCH * D,), jnp.float32)),
    mesh=_mesh,
    compiler_params=pltpu.CompilerParams(
        needs_layout_passes=False, use_tc_tiling_on_sc=True),
    scratch_types=[
        pltpu.VMEM((BATCH,), jnp.int32),
        pltpu.VMEM((NBKT,), jnp.int32),
        pltpu.VMEM((NBKT,), jnp.int32),
        pltpu.VMEM((HITCAP,), jnp.int32),
        pltpu.VMEM((HITCAP,), jnp.int32),
        pltpu.SMEM((HITCAP,), jnp.int32),
        pltpu.SMEM((NBKT + 1,), jnp.int32),
        pltpu.VMEM((12 * D, BLK), jnp.float32),
        pltpu.VMEM((16 * D,), jnp.float32),
        pltpu.SemaphoreType.DMA,
        pltpu.SemaphoreType.DMA,
    ],
)
def _extract_kernel(users_hbm, movies_hbm, utT_hbm, mtT_hbm,
                    uout_hbm, mout_hbm,
                    idxbuf_v, cnt_v, cur_v, hits_v, sorted_v, sorted_s,
                    starts_s, blockbuf_v, stage_v, sem_blk, sem_out):
    wid = lax.axis_index("s") * NUM_CORES + lax.axis_index("c")
    _extract_one(mtT_hbm, movies_hbm, mout_hbm, NMOVIES,
                 idxbuf_v, cnt_v, cur_v, hits_v, sorted_v, sorted_s, starts_s,
                 blockbuf_v, stage_v, sem_blk, sem_out, wid)
    _extract_one(utT_hbm, users_hbm, uout_hbm, NUSERS,
                 idxbuf_v, cnt_v, cur_v, hits_v, sorted_v, sorted_s, starts_s,
                 blockbuf_v, stage_v, sem_blk, sem_out, wid)


TC_TILE = 2048


def _tc_dot_body(u_ref, m_ref, w_ref, o_ref):
    x = u_ref[...] * m_ref[...]
    o_ref[...] = jax.lax.dot_general(
        x, w_ref[...], (((1,), (1,)), ((), ())),
        preferred_element_type=jnp.float32)


_tc_dot = pl.pallas_call(
    _tc_dot_body,
    grid=(BATCH // TC_TILE,),
    in_specs=[
        pl.BlockSpec((TC_TILE, D), lambda i: (i, 0)),
        pl.BlockSpec((TC_TILE, D), lambda i: (i, 0)),
        pl.BlockSpec((1, D), lambda i: (0, 0)),
    ],
    out_specs=pl.BlockSpec((TC_TILE, 1), lambda i: (i, 0)),
    out_shape=jax.ShapeDtypeStruct((BATCH, 1), jnp.float32),
)


def kernel(users, movies, user_table, movie_table, W):
    urows, mrows = _extract_kernel(users, movies, user_table.T, movie_table.T)
    return _tc_dot(urows.reshape(BATCH, D), mrows.reshape(BATCH, D), W)


# speedup vs baseline: 1.0211x; 1.0211x over previous
import functools

import jax
import jax.numpy as jnp
from jax import lax
from jax.experimental import pallas as pl
from jax.experimental.pallas import tpu as pltpu
from jax.experimental.pallas import tpu_sc as plsc

NUM_CORES = 2
NUM_SUBCORES = 16
LANES = 16
NUM_WORKERS = NUM_CORES * NUM_SUBCORES

BATCH = 16384
D = 64
BPW = BATCH // NUM_WORKERS
NVEC = BATCH // LANES
NBKT = 256
HITCAP = 1024
BLK = 128

NUSERS = 1000000
NMOVIES = 100000

_mesh = plsc.VectorSubcoreMesh(core_axis_name="c", subcore_axis_name="s")


def _extract_one(table_hbm, idx_hbm, out_hbm, n_rows,
                 idxbuf_v, cnt_v, cur_v, hits_v, sorted_v, sorted_s, starts_s,
                 blockbuf_v, stage_v, sem_blk, sem_out, wid):
    max_j = (n_rows - 1) // BLK
    nloc = max_j // NUM_WORKERS + 2

    lane = lax.iota(jnp.int32, LANES)
    zeros16 = jnp.zeros((LANES,), jnp.int32)
    ones16 = jnp.ones((LANES,), jnp.int32)

    pltpu.sync_copy(idx_hbm, idxbuf_v)

    for k in range(NBKT // LANES):
        cnt_v[pl.ds(k * LANES, LANES)] = zeros16

    def compact_body(v4, cursor):
        for i in range(4):
            v = v4 * 4 + i
            u = idxbuf_v[pl.ds(v * LANES, LANES)]
            j = lax.shift_right_logical(u, 7)
            own = lax.bitwise_and(j, NUM_WORKERS - 1) == wid
            l = lax.shift_right_logical(j, 5)
            rl = lax.bitwise_and(u, BLK - 1)
            b_vec = lane + v * LANES
            packed = lax.bitwise_or(
                lax.bitwise_or(lax.shift_left(b_vec, 15),
                               lax.shift_left(l, 7)), rl)
            plsc.store_compressed(hits_v.at[pl.ds(cursor, LANES)], packed, mask=own)
            cnt = plsc.all_reduce_population_count(own)
            cursor = cursor + cnt[0]
        return cursor

    total_hits = lax.fori_loop(0, NVEC // 4, compact_body, jnp.int32(0))
    nhv = lax.div(total_hits + (LANES - 1), jnp.int32(LANES))

    def count_body(v, carry):
        p = hits_v[pl.ds(v * LANES, LANES)]
        valid = (lane + v * LANES) < total_hits
        l = lax.bitwise_and(lax.shift_right_logical(p, 7), NBKT - 1)
        rank, last = plsc.scan_count(l, mask=valid)
        plsc.addupdate_scatter(cnt_v, [l], rank,
                               mask=lax.bitwise_and(last, valid))
        return carry

    lax.fori_loop(0, nhv, count_body, 0)

    carry = jnp.int32(0)
    for k in range(NBKT // LANES):
        sl = pl.ds(k * LANES, LANES)
        c = cnt_v[sl]
        cum = plsc.cumsum(c)
        excl = cum - c + carry
        cur_v[sl] = excl
        for j in range(LANES):
            starts_s[k * LANES + j] = excl[j]
        carry = carry + cum[LANES - 1]
    starts_s[NBKT] = total_hits

    def scat_body(v, carry):
        p = hits_v[pl.ds(v * LANES, LANES)]
        valid = (lane + v * LANES) < total_hits
        l = lax.bitwise_and(lax.shift_right_logical(p, 7), NBKT - 1)
        rank, last = plsc.scan_count(l, mask=valid)
        base = plsc.load_gather(cur_v, [l])
        pos = base + rank - 1
        b_vec = lax.shift_right_logical(p, 15)
        p2 = lax.bitwise_or(lax.shift_left(b_vec, 7),
                            lax.bitwise_and(p, BLK - 1))
        plsc.store_scatter(sorted_v, [pos], p2, mask=valid)
        plsc.store_scatter(cur_v, [l], pos + ones16,
                           mask=lax.bitwise_and(last, valid))
        return carry

    lax.fori_loop(0, nhv, scat_body, 0)

    def smem_body(v, carry):
        s = sorted_v[pl.ds(v * LANES, LANES)]
        for j in range(LANES):
            sorted_s[v * LANES + j] = s[j]
        return carry

    lax.fori_loop(0, nhv, smem_body, 0)

    def nonempty(l):
        lo = starts_s[l]
        hi = jnp.where(l + 1 < NBKT, starts_s[l + 1], total_hits)
        return hi > lo

    def fetch(l):
        j = l * NUM_WORKERS + wid
        par = lax.rem(l, 8)

        @pl.when(jnp.logical_and(
            jnp.logical_and(j <= max_j, l < nloc), nonempty(l)))
        def _():
            off = pl.multiple_of(j * BLK, BLK)
            row = pl.multiple_of(par * D, 8)
            pltpu.async_copy(
                table_hbm.at[pl.ds(0, D), pl.ds(off, BLK)],
                blockbuf_v.at[pl.ds(row, D), pl.ds(0, BLK)], sem_blk)

    for p in range(7):
        fetch(jnp.int32(p))

    def sweep_body(l, carry):
        j = l * NUM_WORKERS + wid
        par = lax.rem(l, 8)
        inflight = carry

        @pl.when(jnp.logical_and(j <= max_j, nonempty(l)))
        def _():
            row = pl.multiple_of(par * D, 8)
            pltpu.make_async_copy(
                table_hbm.at[pl.ds(0, D), pl.ds(0, BLK)],
                blockbuf_v.at[pl.ds(row, D), pl.ds(0, BLK)], sem_blk).wait()

        fetch(l + 7)

        def hit_body(h, infl, par=par):
            s = sorted_s[h]
            b = lax.shift_right_logical(s, 7)
            rl = lax.bitwise_and(s, BLK - 1)
            slot = lax.rem(h, 16)

            @pl.when(infl >= 16)
            def _():
                pltpu.make_async_copy(
                    stage_v.at[pl.ds(0, D)], out_hbm.at[pl.ds(0, D)],
                    sem_out).wait()

            rowbase = lane + par * D
            for k in range(D // LANES):
                col = plsc.load_gather(
                    blockbuf_v,
                    [rowbase + k * LANES,
                     jnp.full((LANES,), 0, jnp.int32) + rl])
                stage_v[pl.ds(slot * D + k * LANES, LANES)] = col
            pltpu.async_copy(stage_v.at[pl.ds(slot * D, D)],
                             out_hbm.at[pl.ds(b * D, D)], sem_out)
            return infl + 1

        lo = starts_s[l]
        hi = jnp.where(l + 1 < NBKT, starts_s[l + 1], total_hits)
        hi = jnp.where(j <= max_j, hi, lo)
        inflight = lax.fori_loop(lo, hi, hit_body, inflight)
        return inflight

    inflight = lax.fori_loop(0, nloc, sweep_body, jnp.int32(0))

    def drain_body(i, carry):
        pltpu.make_async_copy(
            stage_v.at[pl.ds(0, D)], out_hbm.at[pl.ds(0, D)], sem_out).wait()
        return carry

    lax.fori_loop(0, lax.min(inflight, jnp.int32(16)), drain_body, 0)


@functools.partial(
    pl.kernel,
    out_type=(jax.ShapeDtypeStruct((BATCH * D,), jnp.float32),
              jax.ShapeDtypeStruct((BATCH * D,), jnp.float32)),
    mesh=_mesh,
    compiler_params=pltpu.CompilerParams(
        needs_layout_passes=False, use_tc_tiling_on_sc=True),
    scratch_types=[
        pltpu.VMEM((BATCH,), jnp.int32),
        pltpu.VMEM((NBKT,), jnp.int32),
        pltpu.VMEM((NBKT,), jnp.int32),
        pltpu.VMEM((HITCAP,), jnp.int32),
        pltpu.VMEM((HITCAP,), jnp.int32),
        pltpu.SMEM((HITCAP,), jnp.int32),
        pltpu.SMEM((NBKT + 1,), jnp.int32),
        pltpu.VMEM((8 * D, BLK), jnp.float32),
        pltpu.VMEM((16 * D,), jnp.float32),
        pltpu.SemaphoreType.DMA,
        pltpu.SemaphoreType.DMA,
    ],
)
def _extract_kernel(users_hbm, movies_hbm, utT_hbm, mtT_hbm,
                    uout_hbm, mout_hbm,
                    idxbuf_v, cnt_v, cur_v, hits_v, sorted_v, sorted_s,
                    starts_s, blockbuf_v, stage_v, sem_blk, sem_out):
    wid = lax.axis_index("s") * NUM_CORES + lax.axis_index("c")
    _extract_one(mtT_hbm, movies_hbm, mout_hbm, NMOVIES,
                 idxbuf_v, cnt_v, cur_v, hits_v, sorted_v, sorted_s, starts_s,
                 blockbuf_v, stage_v, sem_blk, sem_out, wid)
    _extract_one(utT_hbm, users_hbm, uout_hbm, NUSERS,
                 idxbuf_v, cnt_v, cur_v, hits_v, sorted_v, sorted_s, starts_s,
                 blockbuf_v, stage_v, sem_blk, sem_out, wid)


TC_TILE = 2048


def _tc_dot_body(u_ref, m_ref, w_ref, o_ref):
    x = u_ref[...] * m_ref[...]
    o_ref[...] = jax.lax.dot_general(
        x, w_ref[...], (((1,), (1,)), ((), ())),
        preferred_element_type=jnp.float32)


_tc_dot = pl.pallas_call(
    _tc_dot_body,
    grid=(BATCH // TC_TILE,),
    in_specs=[
        pl.BlockSpec((TC_TILE, D), lambda i: (i, 0)),
        pl.BlockSpec((TC_TILE, D), lambda i: (i, 0)),
        pl.BlockSpec((1, D), lambda i: (0, 0)),
    ],
    out_specs=pl.BlockSpec((TC_TILE, 1), lambda i: (i, 0)),
    out_shape=jax.ShapeDtypeStruct((BATCH, 1), jnp.float32),
)


def kernel(users, movies, user_table, movie_table, W):
    urows, mrows = _extract_kernel(users, movies, user_table.T, movie_table.T)
    return _tc_dot(urows.reshape(BATCH, D), mrows.reshape(BATCH, D), W)
